# baseline (speedup 1.0000x reference)
import functools

import jax
import jax.numpy as jnp
from jax import lax
from jax.experimental import pallas as pl
from jax.experimental.pallas import tpu as pltpu
from jax.experimental.pallas import tpu_sc as plsc

EMBED_DIM = 64
PAIR_W = 2 * EMBED_DIM
CTX = 20
LANES = 16
PACK_ROWS = 160


def _pack_pairs(u_weight):
    n_tab = u_weight.shape[0]
    info = plsc.get_sparse_core_info()
    nw = info.num_cores * info.num_subcores
    n_chunks = n_tab // PACK_ROWS
    n_iter = (n_chunks + nw - 1) // nw
    assert n_iter % 2 == 0
    mesh = plsc.VectorSubcoreMesh(core_axis_name="c", subcore_axis_name="s")

    @functools.partial(
        pl.kernel,
        mesh=mesh,
        compiler_params=pltpu.CompilerParams(needs_layout_passes=False),
        out_type=jax.ShapeDtypeStruct((n_tab // 2, PAIR_W), jnp.float32),
        scratch_types=[
            pltpu.VMEM((PACK_ROWS, EMBED_DIM), jnp.float32),
            pltpu.VMEM((PACK_ROWS, EMBED_DIM), jnp.float32),
            pltpu.VMEM((PACK_ROWS // 2, PAIR_W), jnp.float32),
            pltpu.VMEM((PACK_ROWS // 2, PAIR_W), jnp.float32),
            pltpu.SemaphoreType.DMA,
            pltpu.SemaphoreType.DMA,
            pltpu.SemaphoreType.DMA,
            pltpu.SemaphoreType.DMA,
        ],
    )
    def pack(u_hbm, out_hbm, uin0, uin1, up0, up1, si0, si1, so0, so1):
        wid = lax.axis_index("s") * info.num_cores + lax.axis_index("c")

        def fire_in(ci, uin, si):
            c = ci * nw + wid

            @pl.when(c < n_chunks)
            def _():
                pltpu.async_copy(
                    u_hbm.at[pl.ds(c * PACK_ROWS, PACK_ROWS), :], uin, si)

        def step(ci, uin, si, up, so, uin_nxt, si_nxt):
            c = ci * nw + wid
            active = c < n_chunks

            @pl.when(active)
            def _():
                pltpu.make_async_copy(
                    u_hbm.at[pl.ds(0, PACK_ROWS), :], uin, si).wait()

            fire_in(ci + 1, uin_nxt, si_nxt)

            @pl.when(active)
            def _():
                @pl.when(ci >= 2)
                def _():
                    pltpu.make_async_copy(
                        up, out_hbm.at[pl.ds(0, PACK_ROWS // 2)], so).wait()

                for i in range(PACK_ROWS):
                    for k in range(4):
                        up[i // 2,
                           pl.ds((i % 2) * EMBED_DIM + k * LANES, LANES)] = (
                               uin[i, pl.ds(k * LANES, LANES)])
                pltpu.async_copy(
                    up,
                    out_hbm.at[pl.ds(c * (PACK_ROWS // 2), PACK_ROWS // 2)],
                    so)

        fire_in(0, uin0, si0)

        def body(i2, carry):
            step(2 * i2, uin0, si0, up0, so0, uin1, si1)
            step(2 * i2 + 1, uin1, si1, up1, so1, uin0, si0)
            return carry

        lax.fori_loop(0, n_iter // 2, body, 0)

        @pl.when((n_iter - 2) * nw + wid < n_chunks)
        def _():
            pltpu.make_async_copy(
                up0, out_hbm.at[pl.ds(0, PACK_ROWS // 2)], so0).wait()

        @pl.when((n_iter - 1) * nw + wid < n_chunks)
        def _():
            pltpu.make_async_copy(
                up1, out_hbm.at[pl.ds(0, PACK_ROWS // 2)], so1).wait()

    return pack(u_weight)


def _skipgram_sc(center_flat, ctx_flat, v_weight, u_packed, batch):
    info = plsc.get_sparse_core_info()
    nc, ns = info.num_cores, info.num_subcores
    nw = nc * ns
    per_w = batch // nw
    chunk = 32
    n_chunks = per_w // chunk
    nrow = chunk * CTX
    n_gathers = nrow // 128

    mesh = plsc.VectorSubcoreMesh(core_axis_name="c", subcore_axis_name="s")

    @functools.partial(
        pl.kernel,
        mesh=mesh,
        compiler_params=pltpu.CompilerParams(needs_layout_passes=False),
        out_type=jax.ShapeDtypeStruct((batch * CTX,), jnp.float32),
        scratch_types=[
            pltpu.VMEM((chunk,), jnp.int32),
            pltpu.VMEM((nrow,), jnp.int32),
            pltpu.VMEM((nrow,), jnp.int32),
            pltpu.VMEM((chunk, EMBED_DIM), jnp.float32),
            pltpu.VMEM((nrow, PAIR_W), jnp.float32),
            pltpu.VMEM((nrow,), jnp.float32),
            pltpu.SemaphoreType.DMA,
        ],
    )
    def sk(center_hbm, ctx_hbm, v_hbm, u_hbm, out_hbm,
           cidx_v, uidx_v, uidx2_v, vrows, urows, outb, sem):
        wid = lax.axis_index("s") * nc + lax.axis_index("c")

        def chunk_body(g, carry):
            base = wid * per_w + g * chunk
            pltpu.sync_copy(center_hbm.at[pl.ds(base, chunk)], cidx_v)
            pltpu.sync_copy(ctx_hbm.at[pl.ds(base * CTX, nrow)], uidx_v)

            def halve_u(j, bc):
                iv = uidx_v[pl.ds(j * LANES, LANES)]
                uidx2_v[pl.ds(j * LANES, LANES)] = iv >> 1
                return bc

            lax.fori_loop(0, nrow // LANES, halve_u, 0)

            cps = []
            for j in range(n_gathers):
                cps.append(pltpu.async_copy(
                    u_hbm.at[uidx2_v.at[pl.ds(j * 128, 128)]],
                    urows.at[pl.ds(j * 128, 128)],
                    sem))

            def fire_v(jj, bc):
                ivec = cidx_v[pl.ds(jj * LANES, LANES)]
                for k in range(LANES):
                    pltpu.async_copy(
                        v_hbm.at[ivec[k]], vrows.at[jj * LANES + k], sem)
                return bc

            lax.fori_loop(0, chunk // LANES, fire_v, 0)
            for cp in cps:
                cp.wait()
            pltpu.make_async_copy(
                v_hbm.at[pl.ds(0, chunk)], vrows, sem).wait()

            lane = lax.iota(jnp.int32, LANES)

            def grp_body(gi, bc):
                b0 = gi * 4
                vv = [[vrows[b0 + bb, pl.ds(k * LANES, LANES)]
                       for k in range(4)] for bb in range(4)]
                r0 = b0 * CTX
                ov = jnp.zeros((LANES,), jnp.float32)
                upar = uidx_v[pl.ds(r0, LANES)] & 1
                for r in range(4 * CTX):
                    bb = r // CTX
                    row = r0 + r
                    if r % LANES == 0 and r > 0:
                        upar = uidx_v[pl.ds(row, LANES)] & 1
                    up = upar[r % LANES] * EMBED_DIM
                    p = urows[row, pl.ds(up, LANES)] * vv[bb][0]
                    for k in range(1, 4):
                        p += urows[row, pl.ds(up + k * LANES, LANES)] * vv[bb][k]
                    s = jnp.sum(p)
                    ov = jnp.where(lane == (r % LANES), s, ov)
                    if r % LANES == LANES - 1:
                        outb[pl.ds(r0 + (r // LANES) * LANES, LANES)] = ov
                        ov = jnp.zeros((LANES,), jnp.float32)
                return bc

            lax.fori_loop(0, chunk // 4, grp_body, 0)
            pltpu.sync_copy(outb, out_hbm.at[pl.ds(base * CTX, nrow)])
            return carry

        lax.fori_loop(0, n_chunks, chunk_body, 0)

    return sk(center_flat, ctx_flat, v_weight, u_packed)


def kernel(center, contexts_and_negatives, v_weight, u_weight):
    batch = center.shape[0]
    center_flat = center.reshape(batch).astype(jnp.int32)
    ctx_flat = contexts_and_negatives.reshape(batch * CTX).astype(jnp.int32)
    u_packed = u_weight.reshape(u_weight.shape[0] // 2, PAIR_W)
    out = _skipgram_sc(center_flat, ctx_flat, v_weight, u_packed, batch)
    return out.reshape(batch, 1, CTX)


# speedup vs baseline: 1.0013x; 1.0013x over previous
import functools

import jax
import jax.numpy as jnp
from jax import lax
from jax.experimental import pallas as pl
from jax.experimental.pallas import tpu as pltpu
from jax.experimental.pallas import tpu_sc as plsc

EMBED_DIM = 64
PAIR_W = 2 * EMBED_DIM
CTX = 20
LANES = 16
PACK_ROWS = 200
PACK_DEPTH = 3


def _pack_pairs(u_weight):
    n_tab = u_weight.shape[0]
    info = plsc.get_sparse_core_info()
    nw = info.num_cores * info.num_subcores
    n_chunks = n_tab // PACK_ROWS
    n_iter = (n_chunks + nw - 1) // nw
    n_steps = ((n_iter + PACK_DEPTH - 1) // PACK_DEPTH) * PACK_DEPTH
    mesh = plsc.VectorSubcoreMesh(core_axis_name="c", subcore_axis_name="s")

    @functools.partial(
        pl.kernel,
        mesh=mesh,
        compiler_params=pltpu.CompilerParams(needs_layout_passes=False),
        out_type=jax.ShapeDtypeStruct((n_tab // 2, PAIR_W), jnp.float32),
        scratch_types=(
            [pltpu.VMEM((PACK_ROWS, EMBED_DIM), jnp.float32)] * PACK_DEPTH
            + [pltpu.VMEM((PACK_ROWS // 2, PAIR_W), jnp.float32)] * PACK_DEPTH
            + [pltpu.SemaphoreType.DMA] * (2 * PACK_DEPTH)
        ),
    )
    def pack(u_hbm, out_hbm, *scratch):
        uins = scratch[:PACK_DEPTH]
        ups = scratch[PACK_DEPTH:2 * PACK_DEPTH]
        sis = scratch[2 * PACK_DEPTH:3 * PACK_DEPTH]
        sos = scratch[3 * PACK_DEPTH:]
        wid = lax.axis_index("s") * info.num_cores + lax.axis_index("c")

        def fire_in(ci, t):
            c = ci * nw + wid

            @pl.when(c < n_chunks)
            def _():
                pltpu.async_copy(
                    u_hbm.at[pl.ds(c * PACK_ROWS, PACK_ROWS), :],
                    uins[t], sis[t])

        def step(ci, t):
            c = ci * nw + wid
            active = c < n_chunks

            @pl.when(active)
            def _():
                pltpu.make_async_copy(
                    u_hbm.at[pl.ds(0, PACK_ROWS), :], uins[t], sis[t]).wait()

                @pl.when(ci >= PACK_DEPTH)
                def _():
                    pltpu.make_async_copy(
                        ups[t], out_hbm.at[pl.ds(0, PACK_ROWS // 2)],
                        sos[t]).wait()

                for i in range(PACK_ROWS):
                    for k in range(4):
                        ups[t][i // 2,
                               pl.ds((i % 2) * EMBED_DIM + k * LANES,
                                     LANES)] = (
                                         uins[t][i, pl.ds(k * LANES, LANES)])
                pltpu.async_copy(
                    ups[t],
                    out_hbm.at[pl.ds(c * (PACK_ROWS // 2), PACK_ROWS // 2)],
                    sos[t])

            fire_in(ci + PACK_DEPTH, t)

        for t in range(PACK_DEPTH):
            fire_in(t, t)

        def body(i3, carry):
            for t in range(PACK_DEPTH):
                step(PACK_DEPTH * i3 + t, t)
            return carry

        lax.fori_loop(0, n_steps // PACK_DEPTH, body, 0)

        for t in range(PACK_DEPTH):
            pltpu.make_async_copy(
                ups[t], out_hbm.at[pl.ds(0, PACK_ROWS // 2)], sos[t]).wait()

    return pack(u_weight)


def _skipgram_sc(center_flat, ctx_flat, v_weight, u_packed, batch):
    info = plsc.get_sparse_core_info()
    nc, ns = info.num_cores, info.num_subcores
    nw = nc * ns
    per_w = batch // nw
    chunk = 32
    n_chunks = per_w // chunk
    nrow = chunk * CTX
    n_gathers = nrow // 128

    mesh = plsc.VectorSubcoreMesh(core_axis_name="c", subcore_axis_name="s")

    @functools.partial(
        pl.kernel,
        mesh=mesh,
        compiler_params=pltpu.CompilerParams(needs_layout_passes=False),
        out_type=jax.ShapeDtypeStruct((batch * CTX,), jnp.float32),
        scratch_types=[
            pltpu.VMEM((chunk,), jnp.int32),
            pltpu.VMEM((nrow,), jnp.int32),
            pltpu.VMEM((nrow,), jnp.int32),
            pltpu.VMEM((chunk, EMBED_DIM), jnp.float32),
            pltpu.VMEM((nrow, PAIR_W), jnp.float32),
            pltpu.VMEM((nrow,), jnp.float32),
            pltpu.SemaphoreType.DMA,
        ],
    )
    def sk(center_hbm, ctx_hbm, v_hbm, u_hbm, out_hbm,
           cidx_v, uidx_v, uidx2_v, vrows, urows, outb, sem):
        wid = lax.axis_index("s") * nc + lax.axis_index("c")

        def chunk_body(g, carry):
            base = wid * per_w + g * chunk
            pltpu.sync_copy(center_hbm.at[pl.ds(base, chunk)], cidx_v)
            pltpu.sync_copy(ctx_hbm.at[pl.ds(base * CTX, nrow)], uidx_v)

            def halve_u(j, bc):
                iv = uidx_v[pl.ds(j * LANES, LANES)]
                uidx2_v[pl.ds(j * LANES, LANES)] = iv >> 1
                return bc

            lax.fori_loop(0, nrow // LANES, halve_u, 0)

            cps = []
            for j in range(n_gathers):
                cps.append(pltpu.async_copy(
                    u_hbm.at[uidx2_v.at[pl.ds(j * 128, 128)]],
                    urows.at[pl.ds(j * 128, 128)],
                    sem))

            def fire_v(jj, bc):
                ivec = cidx_v[pl.ds(jj * LANES, LANES)]
                for k in range(LANES):
                    pltpu.async_copy(
                        v_hbm.at[ivec[k]], vrows.at[jj * LANES + k], sem)
                return bc

            lax.fori_loop(0, chunk // LANES, fire_v, 0)
            for cp in cps:
                cp.wait()
            pltpu.make_async_copy(
                v_hbm.at[pl.ds(0, chunk)], vrows, sem).wait()

            lane = lax.iota(jnp.int32, LANES)

            def grp_body(gi, bc):
                b0 = gi * 4
                vv = [[vrows[b0 + bb, pl.ds(k * LANES, LANES)]
                       for k in range(4)] for bb in range(4)]
                r0 = b0 * CTX
                ov = jnp.zeros((LANES,), jnp.float32)
                upar = uidx_v[pl.ds(r0, LANES)] & 1
                for r in range(4 * CTX):
                    bb = r // CTX
                    row = r0 + r
                    if r % LANES == 0 and r > 0:
                        upar = uidx_v[pl.ds(row, LANES)] & 1
                    up = upar[r % LANES] * EMBED_DIM
                    p = urows[row, pl.ds(up, LANES)] * vv[bb][0]
                    for k in range(1, 4):
                        p += urows[row, pl.ds(up + k * LANES, LANES)] * vv[bb][k]
                    s = jnp.sum(p)
                    ov = jnp.where(lane == (r % LANES), s, ov)
                    if r % LANES == LANES - 1:
                        outb[pl.ds(r0 + (r // LANES) * LANES, LANES)] = ov
                        ov = jnp.zeros((LANES,), jnp.float32)
                return bc

            lax.fori_loop(0, chunk // 4, grp_body, 0)
            pltpu.sync_copy(outb, out_hbm.at[pl.ds(base * CTX, nrow)])
            return carry

        lax.fori_loop(0, n_chunks, chunk_body, 0)

    return sk(center_flat, ctx_flat, v_weight, u_packed)


def kernel(center, contexts_and_negatives, v_weight, u_weight):
    batch = center.shape[0]
    center_flat = center.reshape(batch).astype(jnp.int32)
    ctx_flat = contexts_and_negatives.reshape(batch * CTX).astype(jnp.int32)
    u_packed = u_weight.reshape(u_weight.shape[0] // 2, PAIR_W)
    out = _skipgram_sc(center_flat, ctx_flat, v_weight, u_packed, batch)
    return out.reshape(batch, 1, CTX)


# speedup vs baseline: 1.2026x; 1.2010x over previous
import functools

import jax
import jax.numpy as jnp
from jax import lax
from jax.experimental import pallas as pl
from jax.experimental.pallas import tpu as pltpu
from jax.experimental.pallas import tpu_sc as plsc

EMBED_DIM = 64
CTX = 20
LANES = 16
NSEM = 8


def _skipgram_sc(center_flat, ctx_flat, v_weight, u_weight, batch):
    info = plsc.get_sparse_core_info()
    nc, ns = info.num_cores, info.num_subcores
    nw = nc * ns
    per_w = batch // nw
    chunk = 32
    n_chunks = per_w // chunk
    nrow = chunk * CTX
    useg = nrow // NSEM

    mesh = plsc.VectorSubcoreMesh(core_axis_name="c", subcore_axis_name="s")

    @functools.partial(
        pl.kernel,
        mesh=mesh,
        compiler_params=pltpu.CompilerParams(needs_layout_passes=False),
        out_type=jax.ShapeDtypeStruct((batch * CTX,), jnp.float32),
        scratch_types=(
            [
                pltpu.VMEM((chunk,), jnp.int32),
                pltpu.VMEM((nrow,), jnp.int32),
                pltpu.VMEM((chunk, EMBED_DIM), jnp.float32),
                pltpu.VMEM((nrow, EMBED_DIM), jnp.float32),
                pltpu.VMEM((nrow,), jnp.float32),
            ]
            + [pltpu.SemaphoreType.DMA] * (NSEM + 1)
        ),
    )
    def sk(center_hbm, ctx_hbm, v_hbm, u_hbm, out_hbm,
           cidx_v, uidx_v, vrows, urows, outb, *sems):
        usems = sems[:NSEM]
        vsem = sems[NSEM]
        wid = lax.axis_index("s") * nc + lax.axis_index("c")

        def chunk_body(g, carry):
            base = wid * per_w + g * chunk
            pltpu.sync_copy(center_hbm.at[pl.ds(base, chunk)], cidx_v)
            pltpu.sync_copy(ctx_hbm.at[pl.ds(base * CTX, nrow)], uidx_v)

            def fire_v(jj, bc):
                ivec = cidx_v[pl.ds(jj * LANES, LANES)]
                for k in range(LANES):
                    pltpu.async_copy(
                        v_hbm.at[ivec[k]], vrows.at[jj * LANES + k], vsem)
                return bc

            def fire_u(jj, bc):
                for s in range(NSEM):
                    base_j = s * useg + jj * LANES
                    ivec = uidx_v[pl.ds(base_j, LANES)]
                    for k in range(LANES):
                        pltpu.async_copy(
                            u_hbm.at[ivec[k]], urows.at[base_j + k], usems[s])
                return bc

            lax.fori_loop(0, chunk // LANES, fire_v, 0)
            lax.fori_loop(0, useg // LANES, fire_u, 0)
            for s in range(NSEM):
                pltpu.make_async_copy(
                    u_hbm.at[pl.ds(0, useg)],
                    urows.at[pl.ds(s * useg, useg)], usems[s]).wait()
            pltpu.make_async_copy(
                v_hbm.at[pl.ds(0, chunk)], vrows, vsem).wait()

            lane = lax.iota(jnp.int32, LANES)

            def grp_body(gi, bc):
                b0 = gi * 4
                vv = [[vrows[b0 + bb, pl.ds(k * LANES, LANES)]
                       for k in range(4)] for bb in range(4)]
                r0 = b0 * CTX
                ov = jnp.zeros((LANES,), jnp.float32)
                for r in range(4 * CTX):
                    bb = r // CTX
                    row = r0 + r
                    p = urows[row, pl.ds(0, LANES)] * vv[bb][0]
                    for k in range(1, 4):
                        p += urows[row, pl.ds(k * LANES, LANES)] * vv[bb][k]
                    s = jnp.sum(p)
                    ov = jnp.where(lane == (r % LANES), s, ov)
                    if r % LANES == LANES - 1:
                        outb[pl.ds(r0 + (r // LANES) * LANES, LANES)] = ov
                        ov = jnp.zeros((LANES,), jnp.float32)
                return bc

            lax.fori_loop(0, chunk // 4, grp_body, 0)
            pltpu.sync_copy(outb, out_hbm.at[pl.ds(base * CTX, nrow)])
            return carry

        lax.fori_loop(0, n_chunks, chunk_body, 0)

    return sk(center_flat, ctx_flat, v_weight, u_weight)


def kernel(center, contexts_and_negatives, v_weight, u_weight):
    batch = center.shape[0]
    center_flat = center.reshape(batch).astype(jnp.int32)
    ctx_flat = contexts_and_negatives.reshape(batch * CTX).astype(jnp.int32)
    out = _skipgram_sc(center_flat, ctx_flat, v_weight, u_weight, batch)
    return out.reshape(batch, 1, CTX)


# speedup vs baseline: 1.2285x; 1.0215x over previous
import functools

import jax
import jax.numpy as jnp
from jax import lax
from jax.experimental import pallas as pl
from jax.experimental.pallas import tpu as pltpu
from jax.experimental.pallas import tpu_sc as plsc

EMBED_DIM = 64
CTX = 20
LANES = 16


def _skipgram_sc(center_flat, ctx_flat, v_weight, u_weight, batch):
    info = plsc.get_sparse_core_info()
    nc, ns = info.num_cores, info.num_subcores
    nw = nc * ns
    per_w = batch // nw
    chunk = 16
    n_chunks = per_w // chunk
    nrow = chunk * CTX
    assert n_chunks % 2 == 0

    mesh = plsc.VectorSubcoreMesh(core_axis_name="c", subcore_axis_name="s")

    @functools.partial(
        pl.kernel,
        mesh=mesh,
        compiler_params=pltpu.CompilerParams(needs_layout_passes=False),
        out_type=jax.ShapeDtypeStruct((batch * CTX,), jnp.float32),
        scratch_types=(
            [pltpu.VMEM((chunk,), jnp.int32)] * 2
            + [pltpu.VMEM((nrow,), jnp.int32)] * 2
            + [pltpu.VMEM((chunk, EMBED_DIM), jnp.float32)] * 2
            + [pltpu.VMEM((nrow, EMBED_DIM), jnp.float32)] * 2
            + [pltpu.VMEM((nrow,), jnp.float32)]
            + [pltpu.SemaphoreType.DMA] * 4
        ),
    )
    def sk(center_hbm, ctx_hbm, v_hbm, u_hbm, out_hbm, *scr):
        cidx = scr[0:2]
        uidx = scr[2:4]
        vrows = scr[4:6]
        urows = scr[6:8]
        outb = scr[8]
        usem = scr[9:11]
        vsem = scr[11:13]
        wid = lax.axis_index("s") * nc + lax.axis_index("c")

        def fire(g, t):
            base = wid * per_w + g * chunk
            pltpu.sync_copy(center_hbm.at[pl.ds(base, chunk)], cidx[t])
            pltpu.sync_copy(ctx_hbm.at[pl.ds(base * CTX, nrow)], uidx[t])

            def fire_v(jj, bc):
                ivec = cidx[t][pl.ds(jj * LANES, LANES)]
                for k in range(LANES):
                    pltpu.async_copy(
                        v_hbm.at[ivec[k]], vrows[t].at[jj * LANES + k],
                        vsem[t])
                return bc

            def fire_u(jj, bc):
                ivec = uidx[t][pl.ds(jj * LANES, LANES)]
                for k in range(LANES):
                    pltpu.async_copy(
                        u_hbm.at[ivec[k]], urows[t].at[jj * LANES + k],
                        usem[t])
                return bc

            lax.fori_loop(0, chunk // LANES, fire_v, 0)
            lax.fori_loop(0, nrow // LANES, fire_u, 0)

        def wait(t):
            pltpu.make_async_copy(
                u_hbm.at[pl.ds(0, nrow)], urows[t], usem[t]).wait()
            pltpu.make_async_copy(
                v_hbm.at[pl.ds(0, chunk)], vrows[t], vsem[t]).wait()

        def compute(g, t):
            base = wid * per_w + g * chunk
            lane = lax.iota(jnp.int32, LANES)

            def grp_body(gi, bc):
                b0 = gi * 4
                vv = [[vrows[t][b0 + bb, pl.ds(k * LANES, LANES)]
                       for k in range(4)] for bb in range(4)]
                r0 = b0 * CTX
                ov = jnp.zeros((LANES,), jnp.float32)
                for r in range(4 * CTX):
                    bb = r // CTX
                    row = r0 + r
                    p = urows[t][row, pl.ds(0, LANES)] * vv[bb][0]
                    for k in range(1, 4):
                        p += urows[t][row, pl.ds(k * LANES, LANES)] * vv[bb][k]
                    s = jnp.sum(p)
                    ov = jnp.where(lane == (r % LANES), s, ov)
                    if r % LANES == LANES - 1:
                        outb[pl.ds(r0 + (r // LANES) * LANES, LANES)] = ov
                        ov = jnp.zeros((LANES,), jnp.float32)
                return bc

            lax.fori_loop(0, chunk // 4, grp_body, 0)
            pltpu.sync_copy(outb, out_hbm.at[pl.ds(base * CTX, nrow)])

        def step(g, t):
            @pl.when(g + 1 < n_chunks)
            def _():
                fire(g + 1, 1 - t)

            wait(t)
            compute(g, t)

        fire(0, 0)

        def body(h, carry):
            step(2 * h, 0)
            step(2 * h + 1, 1)
            return carry

        lax.fori_loop(0, n_chunks // 2, body, 0)

    return sk(center_flat, ctx_flat, v_weight, u_weight)


def kernel(center, contexts_and_negatives, v_weight, u_weight):
    batch = center.shape[0]
    center_flat = center.reshape(batch).astype(jnp.int32)
    ctx_flat = contexts_and_negatives.reshape(batch * CTX).astype(jnp.int32)
    out = _skipgram_sc(center_flat, ctx_flat, v_weight, u_weight, batch)
    return out.reshape(batch, 1, CTX)
